# baseline (speedup 1.0000x reference)
import functools

import jax
import jax.numpy as jnp
from jax.experimental import pallas as pl
from jax.experimental.pallas import tpu as pltpu
from jax.experimental.pallas import tpu_sc as plsc

_W = 65536
_S = 8192
_SUB = _W // _S
_TC_HEAD = 11
_CH = 16384


def _profile(xs):
    w = xs.shape[1]
    while w > 128:
        w //= 2
        xs = jnp.maximum(xs[:, :w], xs[:, w:])
    return xs


def _scan_body(nb, last_blk, n, x_ref, oq_ref, omax_ref, prof_ref):
    i = pl.program_id(0)
    rows = x_ref.shape[0]

    @pl.when(i < nb - 1)
    def _full():
        for k in range(_SUB):
            xs = x_ref[:, k * _S:(k + 1) * _S]
            prof_ref[i * _SUB + k] = _profile(xs)

    @pl.when(i == nb - 1)
    def _tail():
        col = last_blk * _W + jax.lax.broadcasted_iota(
            jnp.int32, x_ref.shape, 1)
        xm = jnp.where(col < n, x_ref[...], -jnp.inf)
        for k in range(_SUB):
            prof_ref[last_blk * _SUB + k] = _profile(
                xm[:, k * _S:(k + 1) * _S])

    @pl.when(i == nb - 1)
    def _fin():
        nq_total = (n + _S - 1) // _S
        qs = list(range(_SUB * (nb - 1))) + list(
            range(last_blk * _SUB, nq_total))
        big = last_blk * _SUB + _SUB
        gprof = prof_ref[qs[0]]
        for q in qs[1:]:
            gprof = jnp.maximum(gprof, prof_ref[q])
        gmax = jnp.max(gprof, axis=1, keepdims=True)
        runq = jnp.full((rows, 128), big, jnp.int32)
        for q in qs:
            runq = jnp.minimum(
                runq, jnp.where(prof_ref[q] == gmax, q, big)
            )
        oq_ref[...] = jnp.min(runq, axis=1, keepdims=True)
        omax_ref[...] = gmax


def _pick_body(n, q_sref, *refs):
    *x_refs, o_ref = refs
    g = pl.program_id(0)
    sub = jax.lax.broadcasted_iota(jnp.int32, (8, 1), 0)
    acc = jnp.zeros((8, 1), jnp.int32)
    for j, x_ref in enumerate(x_refs):
        q = q_sref[8 * g + j]
        xj = x_ref[...]
        col = q * _S + jax.lax.broadcasted_iota(jnp.int32, xj.shape, 1)
        xm = jnp.where(col < n, xj, -jnp.inf)
        bmax = jnp.max(xm, axis=1, keepdims=True)
        lwin = jnp.min(jnp.where(xm == bmax, col, n), axis=1, keepdims=True)
        acc = jnp.where(sub == j, lwin, acc)
    o_ref[...] = acc


def _sc_body(c0, c1, x_hbm, omax_hbm, ocol_hbm,
             buf0, buf1, res_f, res_i, sem0, sem1):
    c = jax.lax.axis_index("c")
    s = jax.lax.axis_index("s")
    w = c * 16 + s
    lane = jax.lax.iota(jnp.int32, 16)
    lane_off = [lane + u * 16 for u in range(8)]
    n_sc = c1 - c0
    starts = list(range(0, n_sc, _CH))
    sizes = [min(_CH, n_sc - st) for st in starts]
    bufs = (buf0, buf1)
    sems = (sem0, sem1)
    for rr in range(2):
        row = w * 2 + rr
        m = jnp.full((16,), -jnp.inf, jnp.float32)
        bc = jnp.zeros((16,), jnp.int32)
        cps = [pltpu.make_async_copy(
            x_hbm.at[row, pl.ds(c0 + starts[0], sizes[0])],
            bufs[0].at[pl.ds(0, sizes[0])], sems[0])]
        cps[0].start()
        for t in range(len(starts)):
            if t + 1 < len(starts):
                nxt = pltpu.make_async_copy(
                    x_hbm.at[row, pl.ds(c0 + starts[t + 1], sizes[t + 1])],
                    bufs[(t + 1) % 2].at[pl.ds(0, sizes[t + 1])],
                    sems[(t + 1) % 2])
                nxt.start()
                cps.append(nxt)
            cps[t].wait()
            buf = bufs[t % 2]
            base = c0 + starts[t]

            def body(kk, carry, buf=buf, base=base):
                m_, bc_ = carry
                p0 = kk * 128
                pairs = []
                for u in range(8):
                    v = buf[pl.ds(p0 + u * 16, 16)]
                    cv = (base + p0) + lane_off[u]
                    pairs.append((v, cv))
                while len(pairs) > 1:
                    nxt_pairs = []
                    for a, b in zip(pairs[0::2], pairs[1::2]):
                        upd = b[0] > a[0]
                        nxt_pairs.append((
                            jnp.where(upd, b[0], a[0]),
                            jnp.where(upd, b[1], a[1]),
                        ))
                    pairs = nxt_pairs
                gv, gc = pairs[0]
                upd = gv > m_
                return jnp.where(upd, gv, m_), jnp.where(upd, gc, bc_)

            m, bc = jax.lax.fori_loop(0, sizes[t] // 128, body, (m, bc))
        res_f[...] = m
        res_i[...] = bc
        pltpu.sync_copy(res_f, omax_hbm.at[row])
        pltpu.sync_copy(res_i, ocol_hbm.at[row])


def kernel(m_logits):
    rows, n = m_logits.shape
    last_blk = (n - 1) // _W
    c0 = _TC_HEAD * _W
    c1 = last_blk * _W
    nb = _TC_HEAD + 1

    sc_mesh = plsc.VectorSubcoreMesh(core_axis_name="c", subcore_axis_name="s")
    sc_lmax, sc_lcol = pl.kernel(
        functools.partial(_sc_body, c0, c1),
        out_type=[
            jax.ShapeDtypeStruct((rows, 16), jnp.float32),
            jax.ShapeDtypeStruct((rows, 16), jnp.int32),
        ],
        mesh=sc_mesh,
        scratch_types=[
            pltpu.VMEM((_CH,), jnp.float32),
            pltpu.VMEM((_CH,), jnp.float32),
            pltpu.VMEM((16,), jnp.float32),
            pltpu.VMEM((16,), jnp.int32),
            pltpu.SemaphoreType.DMA,
            pltpu.SemaphoreType.DMA,
        ],
    )(m_logits)

    qwin, tc_max = pl.pallas_call(
        functools.partial(_scan_body, nb, last_blk, n),
        grid=(nb,),
        in_specs=[pl.BlockSpec(
            (rows, _W),
            lambda i: (0, jnp.where(i < nb - 1, i, last_blk)))],
        out_specs=[
            pl.BlockSpec((rows, 1), lambda i: (0, 0)),
            pl.BlockSpec((rows, 1), lambda i: (0, 0)),
        ],
        out_shape=[
            jax.ShapeDtypeStruct((rows, 1), jnp.int32),
            jax.ShapeDtypeStruct((rows, 1), jnp.float32),
        ],
        scratch_shapes=[
            pltpu.VMEM(((last_blk + 1) * _SUB, rows, 128), jnp.float32),
        ],
    )(m_logits)

    def _in_spec(j):
        return pl.BlockSpec(
            (8, _S), lambda g, q_ref, j=j: (g, q_ref[8 * g + j])
        )

    tc_col = pl.pallas_call(
        functools.partial(_pick_body, n),
        grid_spec=pltpu.PrefetchScalarGridSpec(
            num_scalar_prefetch=1,
            grid=(rows // 8,),
            in_specs=[_in_spec(j) for j in range(8)],
            out_specs=pl.BlockSpec((8, 1), lambda g, q_ref: (g, 0)),
        ),
        out_shape=jax.ShapeDtypeStruct((rows, 1), jnp.int32),
    )(jnp.reshape(qwin, (rows,)), *([m_logits] * 8))

    sc_max = jnp.max(sc_lmax, axis=1, keepdims=True)
    sc_col = jnp.min(
        jnp.where(sc_lmax == sc_max, sc_lcol, n), axis=1, keepdims=True
    )
    tc_in_tail = qwin >= last_blk * _SUB
    sc_wins = (sc_max > tc_max) | ((sc_max == tc_max) & tc_in_tail)
    return jnp.where(sc_wins, sc_col, tc_col)


# speedup vs baseline: 1.0105x; 1.0105x over previous
import functools

import jax
import jax.numpy as jnp
from jax.experimental import pallas as pl
from jax.experimental.pallas import tpu as pltpu
from jax.experimental.pallas import tpu_sc as plsc

_W = 65536
_S = 8192
_SUB = _W // _S
_TC_HEAD = 14
_CH = 16384


def _profile(xs):
    w = xs.shape[1]
    while w > 128:
        w //= 2
        xs = jnp.maximum(xs[:, :w], xs[:, w:])
    return xs


def _scan_body(nb, last_blk, n, x_ref, oq_ref, omax_ref, prof_ref):
    i = pl.program_id(0)
    rows = x_ref.shape[0]

    @pl.when(i < nb - 1)
    def _full():
        for k in range(_SUB):
            xs = x_ref[:, k * _S:(k + 1) * _S]
            prof_ref[i * _SUB + k] = _profile(xs)

    @pl.when(i == nb - 1)
    def _tail():
        col = last_blk * _W + jax.lax.broadcasted_iota(
            jnp.int32, x_ref.shape, 1)
        xm = jnp.where(col < n, x_ref[...], -jnp.inf)
        for k in range(_SUB):
            prof_ref[last_blk * _SUB + k] = _profile(
                xm[:, k * _S:(k + 1) * _S])

    @pl.when(i == nb - 1)
    def _fin():
        nq_total = (n + _S - 1) // _S
        qs = list(range(_SUB * (nb - 1))) + list(
            range(last_blk * _SUB, nq_total))
        big = last_blk * _SUB + _SUB
        gprof = prof_ref[qs[0]]
        for q in qs[1:]:
            gprof = jnp.maximum(gprof, prof_ref[q])
        gmax = jnp.max(gprof, axis=1, keepdims=True)
        runq = jnp.full((rows, 128), big, jnp.int32)
        for q in qs:
            runq = jnp.minimum(
                runq, jnp.where(prof_ref[q] == gmax, q, big)
            )
        oq_ref[...] = jnp.min(runq, axis=1, keepdims=True)
        omax_ref[...] = gmax


def _pick_body(n, q_sref, *refs):
    *x_refs, o_ref = refs
    g = pl.program_id(0)
    sub = jax.lax.broadcasted_iota(jnp.int32, (8, 1), 0)
    acc = jnp.zeros((8, 1), jnp.int32)
    for j, x_ref in enumerate(x_refs):
        q = q_sref[8 * g + j]
        xj = x_ref[...]
        col = q * _S + jax.lax.broadcasted_iota(jnp.int32, xj.shape, 1)
        xm = jnp.where(col < n, xj, -jnp.inf)
        bmax = jnp.max(xm, axis=1, keepdims=True)
        lwin = jnp.min(jnp.where(xm == bmax, col, n), axis=1, keepdims=True)
        acc = jnp.where(sub == j, lwin, acc)
    o_ref[...] = acc


def _sc_body(c0, c1, x_hbm, omax_hbm, ocol_hbm,
             buf0, buf1, res_f, res_i, sem0, sem1):
    c = jax.lax.axis_index("c")
    s = jax.lax.axis_index("s")
    w = c * 16 + s
    lane = jax.lax.iota(jnp.int32, 16)
    lane_off = [lane + u * 16 for u in range(8)]
    n_sc = c1 - c0
    starts = list(range(0, n_sc, _CH))
    sizes = [min(_CH, n_sc - st) for st in starts]
    bufs = (buf0, buf1)
    sems = (sem0, sem1)
    for rr in range(2):
        row = w * 2 + rr
        m = jnp.full((16,), -jnp.inf, jnp.float32)
        bc = jnp.zeros((16,), jnp.int32)
        cps = [pltpu.make_async_copy(
            x_hbm.at[row, pl.ds(c0 + starts[0], sizes[0])],
            bufs[0].at[pl.ds(0, sizes[0])], sems[0])]
        cps[0].start()
        for t in range(len(starts)):
            if t + 1 < len(starts):
                nxt = pltpu.make_async_copy(
                    x_hbm.at[row, pl.ds(c0 + starts[t + 1], sizes[t + 1])],
                    bufs[(t + 1) % 2].at[pl.ds(0, sizes[t + 1])],
                    sems[(t + 1) % 2])
                nxt.start()
                cps.append(nxt)
            cps[t].wait()
            buf = bufs[t % 2]
            base = c0 + starts[t]

            def body(kk, carry, buf=buf, base=base):
                m_, bc_ = carry
                p0 = kk * 128
                pairs = []
                for u in range(8):
                    v = buf[pl.ds(p0 + u * 16, 16)]
                    cv = (base + p0) + lane_off[u]
                    pairs.append((v, cv))
                while len(pairs) > 1:
                    nxt_pairs = []
                    for a, b in zip(pairs[0::2], pairs[1::2]):
                        upd = b[0] > a[0]
                        nxt_pairs.append((
                            jnp.where(upd, b[0], a[0]),
                            jnp.where(upd, b[1], a[1]),
                        ))
                    pairs = nxt_pairs
                gv, gc = pairs[0]
                upd = gv > m_
                return jnp.where(upd, gv, m_), jnp.where(upd, gc, bc_)

            m, bc = jax.lax.fori_loop(0, sizes[t] // 128, body, (m, bc))
        res_f[...] = m
        res_i[...] = bc
        pltpu.sync_copy(res_f, omax_hbm.at[row])
        pltpu.sync_copy(res_i, ocol_hbm.at[row])


def kernel(m_logits):
    rows, n = m_logits.shape
    last_blk = (n - 1) // _W
    c0 = _TC_HEAD * _W
    c1 = last_blk * _W
    nb = _TC_HEAD + 1

    sc_mesh = plsc.VectorSubcoreMesh(core_axis_name="c", subcore_axis_name="s")
    sc_lmax, sc_lcol = pl.kernel(
        functools.partial(_sc_body, c0, c1),
        out_type=[
            jax.ShapeDtypeStruct((rows, 16), jnp.float32),
            jax.ShapeDtypeStruct((rows, 16), jnp.int32),
        ],
        mesh=sc_mesh,
        scratch_types=[
            pltpu.VMEM((_CH,), jnp.float32),
            pltpu.VMEM((_CH,), jnp.float32),
            pltpu.VMEM((16,), jnp.float32),
            pltpu.VMEM((16,), jnp.int32),
            pltpu.SemaphoreType.DMA,
            pltpu.SemaphoreType.DMA,
        ],
    )(m_logits)

    qwin, tc_max = pl.pallas_call(
        functools.partial(_scan_body, nb, last_blk, n),
        grid=(nb,),
        in_specs=[pl.BlockSpec(
            (rows, _W),
            lambda i: (0, jnp.where(i < nb - 1, i, last_blk)))],
        out_specs=[
            pl.BlockSpec((rows, 1), lambda i: (0, 0)),
            pl.BlockSpec((rows, 1), lambda i: (0, 0)),
        ],
        out_shape=[
            jax.ShapeDtypeStruct((rows, 1), jnp.int32),
            jax.ShapeDtypeStruct((rows, 1), jnp.float32),
        ],
        scratch_shapes=[
            pltpu.VMEM(((last_blk + 1) * _SUB, rows, 128), jnp.float32),
        ],
    )(m_logits)

    def _in_spec(j):
        return pl.BlockSpec(
            (8, _S), lambda g, q_ref, j=j: (g, q_ref[8 * g + j])
        )

    tc_col = pl.pallas_call(
        functools.partial(_pick_body, n),
        grid_spec=pltpu.PrefetchScalarGridSpec(
            num_scalar_prefetch=1,
            grid=(rows // 8,),
            in_specs=[_in_spec(j) for j in range(8)],
            out_specs=pl.BlockSpec((8, 1), lambda g, q_ref: (g, 0)),
        ),
        out_shape=jax.ShapeDtypeStruct((rows, 1), jnp.int32),
    )(jnp.reshape(qwin, (rows,)), *([m_logits] * 8))

    sc_max = jnp.max(sc_lmax, axis=1, keepdims=True)
    sc_col = jnp.min(
        jnp.where(sc_lmax == sc_max, sc_lcol, n), axis=1, keepdims=True
    )
    tc_in_tail = qwin >= last_blk * _SUB
    sc_wins = (sc_max > tc_max) | ((sc_max == tc_max) & tc_in_tail)
    return jnp.where(sc_wins, sc_col, tc_col)
